# Initial kernel scaffold; baseline (speedup 1.0000x reference)
import jax
import jax.numpy as jnp
from jax.experimental import pallas as pl


def kernel(item_table, shop_table, user_table, W_u, b_u, item_ids, shop_ids, user_ids):
    raise NotImplementedError("write your pallas kernel here")


# speedup vs baseline: 5.2478x; 5.2478x over previous
import functools

import jax
import jax.numpy as jnp
from jax import lax
from jax.experimental import pallas as pl
from jax.experimental.pallas import tpu as pltpu
from jax.experimental.pallas import tpu_sc as plsc

BATCH = 16384
HIST = 50
DIM = 64
_G = 64
_INV_HIST = 1.0 / HIST


def _sc_gather_kernel(item_hbm, shop_hbm, user_hbm, iid_hbm, sid_hbm, uid_hbm,
                      hist_out, urow_out,
                      idx_i, idx_s, uidx, rows, urows, stage, sem, usem):
    info = plsc.get_sparse_core_info()
    nc = info.num_cores
    wid = lax.axis_index("s") * nc + lax.axis_index("c")
    nw = nc * info.num_subcores
    b_per_w = BATCH // nw
    groups = b_per_w // _G
    b0 = wid * b_per_w

    def accum_rows(_):
        z = jnp.zeros((16,), jnp.float32)

        def body(h, accs):
            a0, a1, a2, a3 = accs
            return (a0 + rows[h, pl.ds(0, 16)],
                    a1 + rows[h, pl.ds(16, 16)],
                    a2 + rows[h, pl.ds(32, 16)],
                    a3 + rows[h, pl.ds(48, 16)])

        return lax.fori_loop(0, HIST, body, (z, z, z, z))

    def group(g, _):
        base = b0 + g * _G
        pltpu.sync_copy(iid_hbm.at[pl.ds(base, _G), :], idx_i)
        pltpu.sync_copy(sid_hbm.at[pl.ds(base, _G), :], idx_s)
        pltpu.sync_copy(uid_hbm.at[pl.ds(base, _G)], uidx)

        def sample(i, _):
            pltpu.async_copy(item_hbm.at[idx_i.at[i]], rows, sem).wait()
            a0, a1, a2, a3 = accum_rows(None)
            stage[i, pl.ds(0, 16)] = a0 * _INV_HIST
            stage[i, pl.ds(16, 16)] = a1 * _INV_HIST
            stage[i, pl.ds(32, 16)] = a2 * _INV_HIST
            stage[i, pl.ds(48, 16)] = a3 * _INV_HIST
            pltpu.async_copy(shop_hbm.at[idx_s.at[i]], rows, sem).wait()
            a0, a1, a2, a3 = accum_rows(None)
            stage[i, pl.ds(64, 16)] = a0 * _INV_HIST
            stage[i, pl.ds(80, 16)] = a1 * _INV_HIST
            stage[i, pl.ds(96, 16)] = a2 * _INV_HIST
            stage[i, pl.ds(112, 16)] = a3 * _INV_HIST
            return 0

        lax.fori_loop(0, _G, sample, 0)
        pltpu.async_copy(user_hbm.at[uidx], urows, usem).wait()
        pltpu.sync_copy(stage, hist_out.at[pl.ds(base, _G), :])
        pltpu.sync_copy(urows, urow_out.at[pl.ds(base, _G), :])
        return 0

    lax.fori_loop(0, groups, group, 0)


def _sc_gather(item_table, shop_table, user_table, item_ids, shop_ids, user_ids):
    mesh = plsc.VectorSubcoreMesh(core_axis_name="c", subcore_axis_name="s")
    fn = functools.partial(
        pl.kernel,
        mesh=mesh,
        out_type=(
            jax.ShapeDtypeStruct((BATCH, 2 * DIM), jnp.float32),
            jax.ShapeDtypeStruct((BATCH, DIM), jnp.float32),
        ),
        scratch_types=[
            pltpu.VMEM((_G, HIST), jnp.int32),
            pltpu.VMEM((_G, HIST), jnp.int32),
            pltpu.VMEM((_G,), jnp.int32),
            pltpu.VMEM((HIST, DIM), jnp.float32),
            pltpu.VMEM((_G, DIM), jnp.float32),
            pltpu.VMEM((_G, 2 * DIM), jnp.float32),
            pltpu.SemaphoreType.DMA,
            pltpu.SemaphoreType.DMA,
        ],
        compiler_params=pltpu.CompilerParams(use_tc_tiling_on_sc=False),
    )(_sc_gather_kernel)
    return fn(item_table, shop_table, user_table, item_ids, shop_ids, user_ids)


def _tc_body(hist_ref, urow_ref, wu_ref, bu_ref, out_ref):
    out_ref[:, : 2 * DIM] = hist_ref[...]
    proj = jnp.dot(urow_ref[...], wu_ref[...],
                   preferred_element_type=jnp.float32)
    out_ref[:, 2 * DIM :] = proj + bu_ref[...]


def _tc_tail(hist, urows, W_u, b_u):
    blk = 2048
    grid = (BATCH // blk,)
    return pl.pallas_call(
        _tc_body,
        grid=grid,
        in_specs=[
            pl.BlockSpec((blk, 2 * DIM), lambda i: (i, 0)),
            pl.BlockSpec((blk, DIM), lambda i: (i, 0)),
            pl.BlockSpec((DIM, DIM), lambda i: (0, 0)),
            pl.BlockSpec((1, DIM), lambda i: (0, 0)),
        ],
        out_specs=pl.BlockSpec((blk, 3 * DIM), lambda i: (i, 0)),
        out_shape=jax.ShapeDtypeStruct((BATCH, 3 * DIM), jnp.float32),
    )(hist, urows, W_u, b_u)


def kernel(item_table, shop_table, user_table, W_u, b_u, item_ids, shop_ids, user_ids):
    item_ids = item_ids.astype(jnp.int32)
    shop_ids = shop_ids.astype(jnp.int32)
    user_ids = user_ids.astype(jnp.int32)
    hist, urows = _sc_gather(item_table, shop_table, user_table,
                             item_ids, shop_ids, user_ids)
    return _tc_tail(hist, urows, W_u, b_u.reshape(1, DIM))
